# baseline (speedup 1.0000x reference)
import functools

import jax
import jax.numpy as jnp
from jax import lax
from jax.experimental import pallas as pl
from jax.experimental.pallas import tpu as pltpu
from jax.experimental.pallas import tpu_sc as plsc

N = 200000
C = 64
SCALE = 4
GLOBAL_SIZE = 64
N_GLOBAL = GLOBAL_SIZE ** 3

NC = 2
NS = 16
NW = NC * NS

B = 128
G = (N + B - 1) // B


def _fusion_body(cx_hbm, cy_hbm, cz_hbm, lf_hbm, gf_hbm, out_hbm,
                 cxv, cyv, czv, idxv, gath, acc, gsem):
    wid = lax.axis_index("s") * NC + lax.axis_index("c")
    cnt = (G + NW - 1 - wid) // NW

    def chunk_body(j, carry):
        g = wid + j * NW
        base = jnp.minimum(g * B, N - B)

        pltpu.sync_copy(cx_hbm.at[pl.ds(base, B)], cxv)
        pltpu.sync_copy(cy_hbm.at[pl.ds(base, B)], cyv)
        pltpu.sync_copy(cz_hbm.at[pl.ds(base, B)], czv)
        pltpu.sync_copy(lf_hbm.at[pl.ds(base, B)], acc)

        for t in range(B // 16):
            sl = pl.ds(t * 16, 16)
            x = jnp.clip(cxv[sl] >> 2, 0, GLOBAL_SIZE - 1)
            y = jnp.clip(cyv[sl] >> 2, 0, GLOBAL_SIZE - 1)
            z = jnp.clip(czv[sl] >> 2, 0, GLOBAL_SIZE - 1)
            idxv[sl] = (x * (GLOBAL_SIZE * GLOBAL_SIZE) + y * GLOBAL_SIZE) + z

        pltpu.async_copy(gf_hbm.at[idxv], gath, gsem).wait()

        def add_row(r, c2):
            for cc in range(C // 16):
                sl = pl.ds(cc * 16, 16)
                acc[r, sl] = acc[r, sl] + gath[r, sl]
            return c2

        lax.fori_loop(0, B, add_row, 0)

        pltpu.sync_copy(acc, out_hbm.at[pl.ds(base, B)])
        return carry

    lax.fori_loop(0, cnt, chunk_body, 0)


@jax.jit
def _fusion(cx, cy, cz, lf, gf):
    mesh = plsc.VectorSubcoreMesh(core_axis_name="c", subcore_axis_name="s")
    return pl.kernel(
        _fusion_body,
        out_type=jax.ShapeDtypeStruct((N, C), jnp.float32),
        mesh=mesh,
        scratch_types=[
            pltpu.VMEM((B,), jnp.int32),
            pltpu.VMEM((B,), jnp.int32),
            pltpu.VMEM((B,), jnp.int32),
            pltpu.VMEM((B,), jnp.int32),
            pltpu.VMEM((B, C), jnp.float32),
            pltpu.VMEM((B, C), jnp.float32),
            pltpu.SemaphoreType.DMA,
        ],
        compiler_params=pltpu.CompilerParams(use_tc_tiling_on_sc=False),
    )(cx, cy, cz, lf, gf)


def kernel(local_features, local_coords, local_base, global_features, global_base):
    adj = (local_coords.astype(jnp.int32)
           + local_base.astype(jnp.int32)[None, :]
           - SCALE * global_base.astype(jnp.int32)[None, :])
    cx = adj[:, 0]
    cy = adj[:, 1]
    cz = adj[:, 2]
    return _fusion(cx, cy, cz, local_features, global_features)


# speedup vs baseline: 1.1813x; 1.1813x over previous
import functools

import jax
import jax.numpy as jnp
from jax import lax
from jax.experimental import pallas as pl
from jax.experimental.pallas import tpu as pltpu
from jax.experimental.pallas import tpu_sc as plsc

N = 200000
C = 64
SCALE = 4
GLOBAL_SIZE = 64
N_GLOBAL = GLOBAL_SIZE ** 3

NC = 2
NS = 16
NW = NC * NS

QB = 128
NQ = 4
B = QB * NQ
G = (N + B - 1) // B


def _fusion_body(cx_hbm, cy_hbm, cz_hbm, lf_hbm, gf_hbm, out_hbm,
                 cxv, cyv, czv, idx0, idx1, idx2, idx3,
                 g0, g1, g2, g3, acc, gsem):
    wid = lax.axis_index("s") * NC + lax.axis_index("c")
    cnt = (G + NW - 1 - wid) // NW
    idxs = (idx0, idx1, idx2, idx3)
    gaths = (g0, g1, g2, g3)

    def chunk_body(j, carry):
        g = wid + j * NW
        base = jnp.minimum(g * B, N - B)

        pltpu.sync_copy(cx_hbm.at[pl.ds(base, B)], cxv)
        pltpu.sync_copy(cy_hbm.at[pl.ds(base, B)], cyv)
        pltpu.sync_copy(cz_hbm.at[pl.ds(base, B)], czv)
        pltpu.sync_copy(lf_hbm.at[pl.ds(base, B)], acc)

        for t in range(B // 16):
            sl = pl.ds(t * 16, 16)
            x = jnp.clip(cxv[sl] >> 2, 0, GLOBAL_SIZE - 1)
            y = jnp.clip(cyv[sl] >> 2, 0, GLOBAL_SIZE - 1)
            z = jnp.clip(czv[sl] >> 2, 0, GLOBAL_SIZE - 1)
            flat = (x * (GLOBAL_SIZE * GLOBAL_SIZE) + y * GLOBAL_SIZE) + z
            idxs[t // (QB // 16)][pl.ds((t % (QB // 16)) * 16, 16)] = flat

        cps = [pltpu.async_copy(gf_hbm.at[idxs[q]], gaths[q], gsem)
               for q in range(NQ)]
        for cp in cps:
            cp.wait()

        for q in range(NQ):
            gq = gaths[q]
            qbase = q * QB

            def add_rows(r, c2, gq=gq, qbase=qbase):
                r4 = qbase + r * 4
                for rr in range(4):
                    for cc in range(C // 16):
                        sl = pl.ds(cc * 16, 16)
                        acc[r4 + rr, sl] = acc[r4 + rr, sl] + gq[r * 4 + rr, sl]
                return c2

            lax.fori_loop(0, QB // 4, add_rows, 0)

        pltpu.sync_copy(acc, out_hbm.at[pl.ds(base, B)])
        return carry

    lax.fori_loop(0, cnt, chunk_body, 0)


@jax.jit
def _fusion(cx, cy, cz, lf, gf):
    mesh = plsc.VectorSubcoreMesh(core_axis_name="c", subcore_axis_name="s")
    return pl.kernel(
        _fusion_body,
        out_type=jax.ShapeDtypeStruct((N, C), jnp.float32),
        mesh=mesh,
        scratch_types=[
            pltpu.VMEM((B,), jnp.int32),
            pltpu.VMEM((B,), jnp.int32),
            pltpu.VMEM((B,), jnp.int32),
            pltpu.VMEM((QB,), jnp.int32),
            pltpu.VMEM((QB,), jnp.int32),
            pltpu.VMEM((QB,), jnp.int32),
            pltpu.VMEM((QB,), jnp.int32),
            pltpu.VMEM((QB, C), jnp.float32),
            pltpu.VMEM((QB, C), jnp.float32),
            pltpu.VMEM((QB, C), jnp.float32),
            pltpu.VMEM((QB, C), jnp.float32),
            pltpu.VMEM((B, C), jnp.float32),
            pltpu.SemaphoreType.DMA,
        ],
        compiler_params=pltpu.CompilerParams(use_tc_tiling_on_sc=False),
    )(cx, cy, cz, lf, gf)


def kernel(local_features, local_coords, local_base, global_features, global_base):
    adj = (local_coords.astype(jnp.int32)
           + local_base.astype(jnp.int32)[None, :]
           - SCALE * global_base.astype(jnp.int32)[None, :])
    cx = adj[:, 0]
    cy = adj[:, 1]
    cz = adj[:, 2]
    return _fusion(cx, cy, cz, local_features, global_features)
